# Initial kernel scaffold; baseline (speedup 1.0000x reference)
import jax
import jax.numpy as jnp
from jax.experimental import pallas as pl


def kernel(input, target, mask):
    raise NotImplementedError("write your pallas kernel here")


# speedup vs baseline: 6.4416x; 6.4416x over previous
import functools
import math

import jax
import jax.numpy as jnp
import numpy as np
from jax.experimental import pallas as pl

SMOOTHING = 0.1
CONFIDENCE = 1.0 - SMOOTHING


def _ls_kernel(x_ref, t_ref, m_ref, acc_ref, msum_ref, *, cols_per_blk,
               fill, conf):
    i = pl.program_id(0)
    j = pl.program_id(1)

    @pl.when((i == 0) & (j == 0))
    def _init():
        acc_ref[...] = jnp.zeros((1, 1), jnp.float32)
        msum_ref[...] = jnp.zeros((1, 1), jnp.float32)

    xb = x_ref[...]
    rows = xb.shape[0]
    tb = t_ref[0, 0, :].reshape(rows, 1)
    mb = m_ref[0, 0, :]

    col0 = j * cols_per_blk
    cols = col0 + jax.lax.broadcasted_iota(jnp.int32, xb.shape, 1)
    onehot = (cols == tb).astype(jnp.float32)
    w = fill + (conf - fill) * onehot
    rowpart = jnp.sum(xb * w, axis=1)
    acc_ref[...] += jnp.sum(rowpart * mb).reshape(1, 1)

    @pl.when(j == 0)
    def _msum():
        msum_ref[...] += jnp.sum(mb).reshape(1, 1)


def kernel(input, target, mask):
    B, T, V = input.shape
    N = B * T
    x = input.reshape(N, V)
    t = target.reshape(N).astype(jnp.int32)
    m = mask.reshape(N).astype(jnp.float32)

    fill = SMOOTHING / (V - 1)
    conf = CONFIDENCE
    c_const = (V - 1) * fill * math.log(fill) + conf * math.log(conf)

    ROWS = 256
    COLS = 4096
    n_i = N // ROWS
    n_j = V // COLS

    t3 = t.reshape(n_i, 1, ROWS)
    m3 = m.reshape(n_i, 1, ROWS)

    body = functools.partial(_ls_kernel, cols_per_blk=COLS,
                             fill=float(np.float32(fill)),
                             conf=float(np.float32(conf)))
    acc, msum = pl.pallas_call(
        body,
        grid=(n_i, n_j),
        in_specs=[
            pl.BlockSpec((ROWS, COLS), lambda i, j: (i, j)),
            pl.BlockSpec((1, 1, ROWS), lambda i, j: (i, 0, 0)),
            pl.BlockSpec((1, 1, ROWS), lambda i, j: (i, 0, 0)),
        ],
        out_specs=[
            pl.BlockSpec((1, 1), lambda i, j: (0, 0)),
            pl.BlockSpec((1, 1), lambda i, j: (0, 0)),
        ],
        out_shape=[
            jax.ShapeDtypeStruct((1, 1), jnp.float32),
            jax.ShapeDtypeStruct((1, 1), jnp.float32),
        ],
    )(x, t3, m3)

    return jnp.float32(c_const) - acc[0, 0] / msum[0, 0]


# speedup vs baseline: 6.9055x; 1.0720x over previous
import functools
import math

import jax
import jax.numpy as jnp
import numpy as np
from jax.experimental import pallas as pl

SMOOTHING = 0.1
CONFIDENCE = 1.0 - SMOOTHING


def _ls_kernel(x_ref, t_ref, m_ref, acc_ref, msum_ref, *, cols_per_blk,
               fill, conf):
    i = pl.program_id(0)
    j = pl.program_id(1)

    @pl.when((i == 0) & (j == 0))
    def _init():
        acc_ref[...] = jnp.zeros((1, 1), jnp.float32)
        msum_ref[...] = jnp.zeros((1, 1), jnp.float32)

    xb = x_ref[...]
    rows = xb.shape[0]
    tb = t_ref[0, 0, :].reshape(rows, 1)
    mb = m_ref[0, 0, :]

    tloc = tb - j * cols_per_blk
    cols = jax.lax.broadcasted_iota(jnp.int32, xb.shape, 1)
    sel = cols == tloc
    rsum = jnp.sum(xb, axis=1)
    gsum = jnp.sum(jnp.where(sel, xb, 0.0), axis=1)
    rowpart = fill * rsum + (conf - fill) * gsum
    acc_ref[...] += jnp.sum(rowpart * mb).reshape(1, 1)

    @pl.when(j == 0)
    def _msum():
        msum_ref[...] += jnp.sum(mb).reshape(1, 1)


def kernel(input, target, mask):
    B, T, V = input.shape
    N = B * T
    x = input.reshape(N, V)
    t = target.reshape(N).astype(jnp.int32)
    m = mask.reshape(N).astype(jnp.float32)

    fill = SMOOTHING / (V - 1)
    conf = CONFIDENCE
    c_const = (V - 1) * fill * math.log(fill) + conf * math.log(conf)

    ROWS = 256
    COLS = 4096
    n_i = N // ROWS
    n_j = V // COLS

    t3 = t.reshape(n_i, 1, ROWS)
    m3 = m.reshape(n_i, 1, ROWS)

    body = functools.partial(_ls_kernel, cols_per_blk=COLS,
                             fill=float(np.float32(fill)),
                             conf=float(np.float32(conf)))
    acc, msum = pl.pallas_call(
        body,
        grid=(n_i, n_j),
        in_specs=[
            pl.BlockSpec((ROWS, COLS), lambda i, j: (i, j)),
            pl.BlockSpec((1, 1, ROWS), lambda i, j: (i, 0, 0)),
            pl.BlockSpec((1, 1, ROWS), lambda i, j: (i, 0, 0)),
        ],
        out_specs=[
            pl.BlockSpec((1, 1), lambda i, j: (0, 0)),
            pl.BlockSpec((1, 1), lambda i, j: (0, 0)),
        ],
        out_shape=[
            jax.ShapeDtypeStruct((1, 1), jnp.float32),
            jax.ShapeDtypeStruct((1, 1), jnp.float32),
        ],
    )(x, t3, m3)

    return jnp.float32(c_const) - acc[0, 0] / msum[0, 0]


# speedup vs baseline: 8.1967x; 1.1870x over previous
import functools
import math

import jax
import jax.numpy as jnp
import numpy as np
from jax.experimental import pallas as pl

SMOOTHING = 0.1
CONFIDENCE = 1.0 - SMOOTHING


def _ls_kernel(x_ref, t_ref, m_ref, acc_ref, msum_ref, *, cols_per_blk,
               fill, conf):
    i = pl.program_id(0)
    j = pl.program_id(1)

    @pl.when((i == 0) & (j == 0))
    def _init():
        acc_ref[...] = jnp.zeros((1, 1), jnp.float32)
        msum_ref[...] = jnp.zeros((1, 1), jnp.float32)

    xb = x_ref[...]
    rows = xb.shape[0]
    tb = t_ref[0, 0, :].reshape(rows, 1)
    mb = m_ref[0, 0, :]

    tloc = tb - j * cols_per_blk
    cols = jax.lax.broadcasted_iota(jnp.int32, xb.shape, 1)
    sel = cols == tloc
    rsum = jnp.sum(xb, axis=1)
    gsum = jnp.sum(jnp.where(sel, xb, 0.0), axis=1)
    rowpart = fill * rsum + (conf - fill) * gsum
    acc_ref[...] += jnp.sum(rowpart * mb).reshape(1, 1)

    @pl.when(j == 0)
    def _msum():
        msum_ref[...] += jnp.sum(mb).reshape(1, 1)


def kernel(input, target, mask):
    B, T, V = input.shape
    N = B * T
    x = input.reshape(N, V)
    t = target.reshape(N).astype(jnp.int32)
    m = mask.reshape(N).astype(jnp.float32)

    fill = SMOOTHING / (V - 1)
    conf = CONFIDENCE
    c_const = (V - 1) * fill * math.log(fill) + conf * math.log(conf)

    ROWS = 512
    COLS = 4096
    n_i = N // ROWS
    n_j = V // COLS

    t3 = t.reshape(n_i, 1, ROWS)
    m3 = m.reshape(n_i, 1, ROWS)

    body = functools.partial(_ls_kernel, cols_per_blk=COLS,
                             fill=float(np.float32(fill)),
                             conf=float(np.float32(conf)))
    acc, msum = pl.pallas_call(
        body,
        grid=(n_i, n_j),
        in_specs=[
            pl.BlockSpec((ROWS, COLS), lambda i, j: (i, j)),
            pl.BlockSpec((1, 1, ROWS), lambda i, j: (i, 0, 0)),
            pl.BlockSpec((1, 1, ROWS), lambda i, j: (i, 0, 0)),
        ],
        out_specs=[
            pl.BlockSpec((1, 1), lambda i, j: (0, 0)),
            pl.BlockSpec((1, 1), lambda i, j: (0, 0)),
        ],
        out_shape=[
            jax.ShapeDtypeStruct((1, 1), jnp.float32),
            jax.ShapeDtypeStruct((1, 1), jnp.float32),
        ],
    )(x, t3, m3)

    return jnp.float32(c_const) - acc[0, 0] / msum[0, 0]


# speedup vs baseline: 8.6117x; 1.0506x over previous
import functools
import math

import jax
import jax.numpy as jnp
import numpy as np
from jax.experimental import pallas as pl

SMOOTHING = 0.1
CONFIDENCE = 1.0 - SMOOTHING


def _ls_kernel(x_ref, t_ref, m_ref, acc_ref, msum_ref, *, cols_per_blk,
               fill, conf):
    i = pl.program_id(0)
    j = pl.program_id(1)

    @pl.when((i == 0) & (j == 0))
    def _init():
        acc_ref[...] = jnp.zeros((1, 1), jnp.float32)
        msum_ref[...] = jnp.zeros((1, 1), jnp.float32)

    xb = x_ref[...]
    rows = xb.shape[0]
    tb = t_ref[0, 0, :].reshape(rows, 1)
    mb = m_ref[0, 0, :]

    tloc = tb - j * cols_per_blk
    cols = jax.lax.broadcasted_iota(jnp.int32, xb.shape, 1)
    sel = cols == tloc
    rsum = jnp.sum(xb, axis=1)
    gsum = jnp.sum(jnp.where(sel, xb, 0.0), axis=1)
    rowpart = fill * rsum + (conf - fill) * gsum
    acc_ref[...] += jnp.sum(rowpart * mb).reshape(1, 1)

    @pl.when(j == 0)
    def _msum():
        msum_ref[...] += jnp.sum(mb).reshape(1, 1)


def kernel(input, target, mask):
    B, T, V = input.shape
    N = B * T
    x = input.reshape(N, V)
    t = target.reshape(N).astype(jnp.int32)
    m = mask.reshape(N).astype(jnp.float32)

    fill = SMOOTHING / (V - 1)
    conf = CONFIDENCE
    c_const = (V - 1) * fill * math.log(fill) + conf * math.log(conf)

    ROWS = 1024
    COLS = 4096
    n_i = N // ROWS
    n_j = V // COLS

    t3 = t.reshape(n_i, 1, ROWS)
    m3 = m.reshape(n_i, 1, ROWS)

    body = functools.partial(_ls_kernel, cols_per_blk=COLS,
                             fill=float(np.float32(fill)),
                             conf=float(np.float32(conf)))
    acc, msum = pl.pallas_call(
        body,
        grid=(n_i, n_j),
        in_specs=[
            pl.BlockSpec((ROWS, COLS), lambda i, j: (i, j)),
            pl.BlockSpec((1, 1, ROWS), lambda i, j: (i, 0, 0)),
            pl.BlockSpec((1, 1, ROWS), lambda i, j: (i, 0, 0)),
        ],
        out_specs=[
            pl.BlockSpec((1, 1), lambda i, j: (0, 0)),
            pl.BlockSpec((1, 1), lambda i, j: (0, 0)),
        ],
        out_shape=[
            jax.ShapeDtypeStruct((1, 1), jnp.float32),
            jax.ShapeDtypeStruct((1, 1), jnp.float32),
        ],
    )(x, t3, m3)

    return jnp.float32(c_const) - acc[0, 0] / msum[0, 0]
